# baseline (device time: 17843 ns/iter reference)
import jax
import jax.numpy as jnp
from jax import lax
from jax.experimental import pallas as pl
from jax.experimental.pallas import tpu as pltpu

_CHUNKS = 8


def kernel(x, dy, gamma):
    del gamma
    m, d = x.shape
    rows = m // 4
    crows = rows // _CHUNKS

    def body(
        x_hbm, dy_hbm, out_ref,
        xb, dyb, partial_ref, recv_ref,
        copy_sems, send_sems, recv_sems,
    ):
        mx = lax.axis_index("x")
        my = lax.axis_index("y")
        mz = lax.axis_index("z")
        off = my * rows

        cps = []
        for c in range(_CHUNKS):
            cp_x = pltpu.make_async_copy(
                x_hbm.at[pl.ds(off + c * crows, crows), :],
                xb.at[pl.ds(c * crows, crows), :],
                copy_sems.at[0, c],
            )
            cp_dy = pltpu.make_async_copy(
                dy_hbm.at[pl.ds(off + c * crows, crows), :],
                dyb.at[pl.ds(c * crows, crows), :],
                copy_sems.at[1, c],
            )
            cp_x.start()
            cp_dy.start()
            cps.append((cp_x, cp_dy))

        peers = [
            (mx ^ dx, my ^ dyy, mz)
            for dx in (0, 1)
            for dyy in (0, 1, 2, 3)
            if (dx, dyy) != (0, 0)
        ]

        barrier_sem = pltpu.get_barrier_semaphore()
        for peer in peers:
            pl.semaphore_signal(
                barrier_sem, inc=1, device_id=peer,
                device_id_type=pl.DeviceIdType.MESH,
            )

        dgamma = None
        dbeta = None
        for c in range(_CHUNKS):
            cp_x, cp_dy = cps[c]
            cp_x.wait()
            cp_dy.wait()
            xv = xb[pl.ds(c * crows, crows), :]
            dyv = dyb[pl.ds(c * crows, crows), :]
            mu = jnp.mean(xv, axis=1, keepdims=True)
            var = jnp.mean((xv - mu) * (xv - mu), axis=1, keepdims=True)
            rstd = lax.rsqrt(var + 1e-5)
            xhat = (xv - mu) * rstd
            dg = jnp.sum(dyv * xhat, axis=0)
            db = jnp.sum(dyv, axis=0)
            dgamma = dg if dgamma is None else dgamma + dg
            dbeta = db if dbeta is None else dbeta + db
        partial_ref[0, :] = dgamma
        partial_ref[1, :] = dbeta

        pl.semaphore_wait(barrier_sem, len(peers))

        rdmas = []
        for i, peer in enumerate(peers):
            rdma = pltpu.make_async_remote_copy(
                src_ref=partial_ref,
                dst_ref=recv_ref.at[i],
                send_sem=send_sems.at[i],
                recv_sem=recv_sems.at[i],
                device_id=peer,
                device_id_type=pl.DeviceIdType.MESH,
            )
            rdma.start()
            rdmas.append(rdma)

        acc = partial_ref[:, :]
        for i, rdma in enumerate(rdmas):
            rdma.wait_recv()
            acc = acc + recv_ref[i]
        out_ref[:, :] = acc
        for rdma in rdmas:
            rdma.wait_send()

    return pl.pallas_call(
        body,
        out_shape=jax.ShapeDtypeStruct((2, d), jnp.float32),
        in_specs=[
            pl.BlockSpec(memory_space=pl.ANY),
            pl.BlockSpec(memory_space=pl.ANY),
        ],
        out_specs=pl.BlockSpec(memory_space=pltpu.VMEM),
        scratch_shapes=[
            pltpu.VMEM((rows, d), jnp.float32),
            pltpu.VMEM((rows, d), jnp.float32),
            pltpu.VMEM((2, d), jnp.float32),
            pltpu.VMEM((7, 2, d), jnp.float32),
            pltpu.SemaphoreType.DMA((2, _CHUNKS)),
            pltpu.SemaphoreType.DMA((7,)),
            pltpu.SemaphoreType.DMA((7,)),
        ],
        compiler_params=pltpu.CompilerParams(collective_id=0),
    )(x, dy)


# device time: 16237 ns/iter; 1.0989x vs baseline; 1.0989x over previous
import jax
import jax.numpy as jnp
from jax import lax
from jax.experimental import pallas as pl
from jax.experimental.pallas import tpu as pltpu

_CHUNKS = 4


def kernel(x, dy, gamma):
    del gamma
    m, d = x.shape
    rows = m // 2
    crows = rows // _CHUNKS

    def body(
        x_hbm, dy_hbm, out_ref,
        xb, dyb, partial_ref, recv_ref,
        copy_sems, send_sems, recv_sems,
    ):
        mx = lax.axis_index("x")
        my = lax.axis_index("y")
        mz = lax.axis_index("z")
        off = (my & 1) * rows

        cps = []
        for c in range(_CHUNKS):
            cp_x = pltpu.make_async_copy(
                x_hbm.at[pl.ds(off + c * crows, crows), :],
                xb.at[pl.ds(c * crows, crows), :],
                copy_sems.at[0, c],
            )
            cp_dy = pltpu.make_async_copy(
                dy_hbm.at[pl.ds(off + c * crows, crows), :],
                dyb.at[pl.ds(c * crows, crows), :],
                copy_sems.at[1, c],
            )
            cp_x.start()
            cp_dy.start()
            cps.append((cp_x, cp_dy))

        peers = [
            (mx, my ^ 1, mz),
            (1 - mx, my, mz),
            (1 - mx, my ^ 1, mz),
        ]

        barrier_sem = pltpu.get_barrier_semaphore()
        for peer in peers:
            pl.semaphore_signal(
                barrier_sem, inc=1, device_id=peer,
                device_id_type=pl.DeviceIdType.MESH,
            )

        dgamma = None
        dbeta = None
        for c in range(_CHUNKS):
            cp_x, cp_dy = cps[c]
            cp_x.wait()
            cp_dy.wait()
            xv = xb[pl.ds(c * crows, crows), :]
            dyv = dyb[pl.ds(c * crows, crows), :]
            mu = jnp.mean(xv, axis=1, keepdims=True)
            var = jnp.mean((xv - mu) * (xv - mu), axis=1, keepdims=True)
            rstd = lax.rsqrt(var + 1e-5)
            xhat = (xv - mu) * rstd
            dg = jnp.sum(dyv * xhat, axis=0)
            db = jnp.sum(dyv, axis=0)
            dgamma = dg if dgamma is None else dgamma + dg
            dbeta = db if dbeta is None else dbeta + db
        partial_ref[0, :] = dgamma
        partial_ref[1, :] = dbeta

        pl.semaphore_wait(barrier_sem, len(peers))

        rdmas = []
        for i, peer in enumerate(peers):
            rdma = pltpu.make_async_remote_copy(
                src_ref=partial_ref,
                dst_ref=recv_ref.at[i],
                send_sem=send_sems.at[i],
                recv_sem=recv_sems.at[i],
                device_id=peer,
                device_id_type=pl.DeviceIdType.MESH,
            )
            rdma.start()
            rdmas.append(rdma)

        acc = partial_ref[:, :]
        for i, rdma in enumerate(rdmas):
            rdma.wait_recv()
            acc = acc + recv_ref[i]
        out_ref[:, :] = acc
        for rdma in rdmas:
            rdma.wait_send()

    return pl.pallas_call(
        body,
        out_shape=jax.ShapeDtypeStruct((2, d), jnp.float32),
        in_specs=[
            pl.BlockSpec(memory_space=pl.ANY),
            pl.BlockSpec(memory_space=pl.ANY),
        ],
        out_specs=pl.BlockSpec(memory_space=pltpu.VMEM),
        scratch_shapes=[
            pltpu.VMEM((rows, d), jnp.float32),
            pltpu.VMEM((rows, d), jnp.float32),
            pltpu.VMEM((2, d), jnp.float32),
            pltpu.VMEM((3, 2, d), jnp.float32),
            pltpu.SemaphoreType.DMA((2, _CHUNKS)),
            pltpu.SemaphoreType.DMA((3,)),
            pltpu.SemaphoreType.DMA((3,)),
        ],
        compiler_params=pltpu.CompilerParams(collective_id=0),
    )(x, dy)
